# Initial kernel scaffold; baseline (speedup 1.0000x reference)
import jax
import jax.numpy as jnp
from jax.experimental import pallas as pl


def kernel(x, edge_index, edge_attr, batch, atom_tab, bond_tab1, bond_tab2, W1, b1, root1, W2, b2, root2, Wg, bg):
    raise NotImplementedError("write your pallas kernel here")


# speedup vs baseline: 1.0001x; 1.0001x over previous
import jax
import jax.numpy as jnp
from jax.experimental import pallas as pl
from jax.experimental.pallas import tpu as pltpu
from jax.experimental.pallas import tpu_sc as plsc

N = 10000
E = 160000
D = 256
G = 64


def _gcn_layer(h, row, col, ea, W, b, root, bond_tab, deg, dis):
    hx = h @ W + b
    ee = bond_tab[0][ea[:, 0]] + bond_tab[1][ea[:, 1]] + bond_tab[2][ea[:, 2]]
    norm = dis[row] * dis[col]
    msg = norm[:, None] * jax.nn.relu(hx[row] + ee)
    agg = jax.ops.segment_sum(msg, col, num_segments=N)
    return agg + jax.nn.relu(hx + root) / deg[:, None]


def kernel(x, edge_index, edge_attr, batch, atom_tab, bond_tab1, bond_tab2,
           W1, b1, root1, W2, b2, root2, Wg, bg):
    row = edge_index[0]
    col = edge_index[1]
    h = atom_tab[0][x[:, 0]]
    for i in range(1, 9):
        h = h + atom_tab[i][x[:, i]]
    deg = jax.ops.segment_sum(jnp.ones((E,), jnp.float32), row, num_segments=N) + 1.0
    dis = deg ** -0.5
    h = _gcn_layer(h, row, col, edge_attr, W1, b1, root1, bond_tab1, deg, dis)
    h = _gcn_layer(h, row, col, edge_attr, W2, b2, root2, bond_tab2, deg, dis)
    hg = jax.ops.segment_max(h, batch, num_segments=G)
    return hg @ Wg + bg


# speedup vs baseline: 3.5310x; 3.5308x over previous
import functools

import jax
import jax.numpy as jnp
from jax import lax
from jax.experimental import pallas as pl
from jax.experimental.pallas import tpu as pltpu
from jax.experimental.pallas import tpu_sc as plsc

N = 10000
E = 160000
D = 256
G = 64

NSC = 2
NTILES = 16
NW = NSC * NTILES
NPAD = 10240
EPT = E // NW

_MESH = dict(core_axis_name="c", subcore_axis_name="s")


def _fill(ref, start, count16, vec):
    def body(i, _):
        ref[pl.ds(start + i * 16, 16)] = vec
        return 0
    lax.fori_loop(0, count16, body, 0)


def _deg_body(row_hbm, out_hbm, idx_v, val_v, zb_v, deg_sh):
    c = lax.axis_index("c")
    s = lax.axis_index("s")
    w = c * NTILES + s
    zsl = NPAD // NTILES
    _fill(zb_v, 0, zsl // 16, jnp.zeros((16,), jnp.float32))
    pltpu.sync_copy(zb_v, deg_sh.at[pl.ds(s * zsl, zsl)])
    _fill(val_v, 0, EPT // 16 + 1, jnp.ones((16,), jnp.float32))
    pltpu.sync_copy(row_hbm.at[pl.ds(w * EPT, EPT)], idx_v)
    plsc.subcore_barrier()
    pltpu.sync_copy(val_v.at[pl.ds(0, EPT)], deg_sh.at[idx_v], add=True)
    plsc.subcore_barrier()
    pltpu.sync_copy(deg_sh.at[pl.ds(s * zsl, zsl)], out_hbm.at[c, pl.ds(s * zsl, zsl)])


CE = 128
EPTP = 10240
NCH = EPTP // CE
DH = D // NSC


def _edge_body(row_hbm, col_hbm, eid_hbm, hx_hbm, ctab_hbm, dis_hbm, out_hbm,
               rowb, colb, eidb, nrmb, hxb, eeb, agg_sh):
    c = lax.axis_index("c")
    s = lax.axis_index("s")
    zsl = NPAD // NTILES
    z16 = jnp.zeros((16,), jnp.float32)

    def zfill(t, _):
        for k in range(DH // 16):
            hxb[t, pl.ds(k * 16, 16)] = z16
        return 0
    lax.fori_loop(0, CE, zfill, 0)

    def zcp(t, _):
        pltpu.sync_copy(hxb, agg_sh.at[pl.ds(s * zsl + t * CE, CE)])
        return 0
    lax.fori_loop(0, zsl // CE, zcp, 0)
    plsc.subcore_barrier()

    def chunk(i, _):
        base = s * EPTP + i * CE
        pltpu.sync_copy(row_hbm.at[pl.ds(base, CE)], rowb)
        pltpu.sync_copy(col_hbm.at[pl.ds(base, CE)], colb)
        pltpu.sync_copy(eid_hbm.at[pl.ds(base, CE)], eidb)
        pltpu.sync_copy(hx_hbm.at[c].at[rowb], hxb)
        pltpu.sync_copy(ctab_hbm.at[c].at[eidb], eeb)

        pltpu.sync_copy(dis_hbm.at[rowb], nrmb)

        def egrp(g, _):
            nv16 = nrmb[pl.ds(g * 16, 16)]
            for jj in range(16):
                j = g * 16 + jj
                nv = nv16[jj]
                for k in range(DH // 16):
                    sl = pl.ds(k * 16, 16)
                    v = hxb[j, sl] + nv * eeb[j, sl]
                    hxb[j, sl] = jnp.maximum(v, 0.0)
            return 0
        lax.fori_loop(0, CE // 16, egrp, 0)
        pltpu.sync_copy(hxb, agg_sh.at[colb], add=True)
        return 0

    lax.fori_loop(0, NCH, chunk, 0)
    plsc.subcore_barrier()
    pltpu.sync_copy(agg_sh.at[pl.ds(s * zsl, zsl)],
                    out_hbm.at[c, pl.ds(s * zsl, zsl)])


@jax.jit
def _edge_call(row, col, eid, hx3, ctab3, dis):
    mesh = plsc.VectorSubcoreMesh(**_MESH)
    f = pl.kernel(
        _edge_body,
        out_type=jax.ShapeDtypeStruct((NSC, NPAD, DH), jnp.float32),
        mesh=mesh,
        scratch_types=[
            pltpu.VMEM((CE,), jnp.int32),
            pltpu.VMEM((CE,), jnp.int32),
            pltpu.VMEM((CE,), jnp.int32),
            pltpu.VMEM((CE,), jnp.float32),
            pltpu.VMEM((CE, DH), jnp.float32),
            pltpu.VMEM((CE, DH), jnp.float32),
            pltpu.VMEM_SHARED((NPAD, DH), jnp.float32),
        ],
    )
    return f(row, col, eid, hx3, ctab3, dis)


@jax.jit
def _deg_call(row):
    mesh = plsc.VectorSubcoreMesh(**_MESH)
    f = pl.kernel(
        _deg_body,
        out_type=jax.ShapeDtypeStruct((NSC, NPAD), jnp.float32),
        mesh=mesh,
        scratch_types=[
            pltpu.VMEM((EPT,), jnp.int32),
            pltpu.VMEM((EPT + 16,), jnp.float32),
            pltpu.VMEM((NPAD // NTILES,), jnp.float32),
            pltpu.VMEM_SHARED((NPAD,), jnp.float32),
        ],
    )
    return f(row)


def _padE(a, fill):
    a2 = a.reshape(NTILES, E // NTILES).astype(jnp.int32)
    return jnp.pad(a2, ((0, 0), (0, EPTP - E // NTILES)),
                   constant_values=fill).reshape(-1)


def _gcn_layer(h, row, col, eid, W, b, root, ctab, deg, dis):
    hx = h @ W + b
    hxp = jnp.zeros((NPAD, D), jnp.float32).at[:N].set(dis[:, None] * hx)
    hx3 = jnp.stack([hxp[:, :DH], hxp[:, DH:]])
    ctab3 = jnp.stack([ctab[:, :DH], ctab[:, DH:]])
    disp = jnp.zeros((NPAD,), jnp.float32).at[:N].set(dis)
    aggp = _edge_call(_padE(row, 0), _padE(col, N), _padE(eid, 0), hx3, ctab3, disp)
    agg = jnp.concatenate([aggp[0, :N], aggp[1, :N]], axis=1) * dis[:, None]
    return agg + jax.nn.relu(hx + root) / deg[:, None]


def kernel(x, edge_index, edge_attr, batch, atom_tab, bond_tab1, bond_tab2,
           W1, b1, root1, W2, b2, root2, Wg, bg):
    row = edge_index[0]
    col = edge_index[1]
    h = atom_tab[0][x[:, 0]]
    for i in range(1, 9):
        h = h + atom_tab[i][x[:, i]]
    degp = _deg_call(row)
    deg = degp[0, :N] + degp[1, :N] + 1.0
    dis = deg ** -0.5
    eid = (edge_attr[:, 0] + 8 * edge_attr[:, 1] + 64 * edge_attr[:, 2]).astype(jnp.int32)
    ids = jnp.arange(512, dtype=jnp.int32)
    def _ctab(bt):
        return bt[0][ids & 7] + bt[1][(ids >> 3) & 7] + bt[2][(ids >> 6) & 7]
    h = _gcn_layer(h, row, col, eid, W1, b1, root1, _ctab(bond_tab1), deg, dis)
    h = _gcn_layer(h, row, col, eid, W2, b2, root2, _ctab(bond_tab2), deg, dis)
    hg = jax.ops.segment_max(h, batch, num_segments=G)
    return hg @ Wg + bg


# speedup vs baseline: 4.5012x; 1.2748x over previous
import functools

import jax
import jax.numpy as jnp
from jax import lax
from jax.experimental import pallas as pl
from jax.experimental.pallas import tpu as pltpu
from jax.experimental.pallas import tpu_sc as plsc

N = 10000
E = 160000
D = 256
G = 64

NSC = 2
NTILES = 16
NW = NSC * NTILES
NPAD = 10240
EPT = E // NW

_MESH = dict(core_axis_name="c", subcore_axis_name="s")


def _fill(ref, start, count16, vec):
    def body(i, _):
        ref[pl.ds(start + i * 16, 16)] = vec
        return 0
    lax.fori_loop(0, count16, body, 0)


def _deg_body(row_hbm, out_hbm, idx_v, val_v, zb_v, deg_sh):
    c = lax.axis_index("c")
    s = lax.axis_index("s")
    w = c * NTILES + s
    zsl = NPAD // NTILES
    _fill(zb_v, 0, zsl // 16, jnp.zeros((16,), jnp.float32))
    pltpu.sync_copy(zb_v, deg_sh.at[pl.ds(s * zsl, zsl)])
    _fill(val_v, 0, EPT // 16 + 1, jnp.ones((16,), jnp.float32))
    pltpu.sync_copy(row_hbm.at[pl.ds(w * EPT, EPT)], idx_v)
    plsc.subcore_barrier()
    pltpu.sync_copy(val_v.at[pl.ds(0, EPT)], deg_sh.at[idx_v], add=True)
    plsc.subcore_barrier()
    pltpu.sync_copy(deg_sh.at[pl.ds(s * zsl, zsl)], out_hbm.at[c, pl.ds(s * zsl, zsl)])


CE = 80
EPTP = 10240
NCH = EPTP // CE
DH = D // NSC


def _edge_body(row_hbm, col_hbm, eid_hbm, hx_hbm, ctab_hbm, dis_hbm, out_hbm,
               rowb, colb, eidb, nrmb, hxb, eeb, agg_sh, sem0, sem1):
    c = lax.axis_index("c")
    s = lax.axis_index("s")
    sems = (sem0, sem1)
    zsl = NPAD // NTILES
    z16 = jnp.zeros((16,), jnp.float32)

    def zfill(t, _):
        for k in range(DH // 16):
            hxb[0, t, pl.ds(k * 16, 16)] = z16
        return 0
    lax.fori_loop(0, CE, zfill, 0)

    def zcp(t, _):
        pltpu.sync_copy(hxb.at[0], agg_sh.at[pl.ds(s * zsl + t * CE, CE)])
        return 0
    lax.fori_loop(0, zsl // CE, zcp, 0)
    plsc.subcore_barrier()

    def fire(b, i):
        base = s * EPTP + i * CE
        pltpu.sync_copy(row_hbm.at[pl.ds(base, CE)], rowb.at[b])
        pltpu.sync_copy(col_hbm.at[pl.ds(base, CE)], colb.at[b])
        pltpu.sync_copy(eid_hbm.at[pl.ds(base, CE)], eidb.at[b])
        pltpu.async_copy(hx_hbm.at[c].at[rowb.at[b]], hxb.at[b], sems[b])
        pltpu.async_copy(ctab_hbm.at[c].at[eidb.at[b]], eeb.at[b], sems[b])
        pltpu.async_copy(dis_hbm.at[rowb.at[b]], nrmb.at[b], sems[b])

    def drain(b):
        pltpu.make_async_copy(hx_hbm.at[c].at[rowb.at[b]], hxb.at[b], sems[b]).wait()
        pltpu.make_async_copy(ctab_hbm.at[c].at[eidb.at[b]], eeb.at[b], sems[b]).wait()
        pltpu.make_async_copy(dis_hbm.at[rowb.at[b]], nrmb.at[b], sems[b]).wait()

    def compute_scatter(b):
        def egrp(g, _):
            nv16 = nrmb[b, pl.ds(g * 16, 16)]
            for jj in range(16):
                j = g * 16 + jj
                nv = nv16[jj]
                for k in range(DH // 16):
                    sl = pl.ds(k * 16, 16)
                    v = hxb[b, j, sl] + nv * eeb[b, j, sl]
                    hxb[b, j, sl] = jnp.maximum(v, 0.0)
            return 0
        lax.fori_loop(0, CE // 16, egrp, 0)
        pltpu.sync_copy(hxb.at[b], agg_sh.at[colb.at[b]], add=True)

    fire(0, 0)

    def chunkpair(g, _):
        i0 = 2 * g
        fire(1, i0 + 1)
        drain(0)
        compute_scatter(0)

        @pl.when(g < NCH // 2 - 1)
        def _():
            fire(0, i0 + 2)
        drain(1)
        compute_scatter(1)
        return 0

    lax.fori_loop(0, NCH // 2, chunkpair, 0)
    plsc.subcore_barrier()
    pltpu.sync_copy(agg_sh.at[pl.ds(s * zsl, zsl)],
                    out_hbm.at[c, pl.ds(s * zsl, zsl)])


@jax.jit
def _edge_call(row, col, eid, hx3, ctab3, dis):
    mesh = plsc.VectorSubcoreMesh(**_MESH)
    f = pl.kernel(
        _edge_body,
        out_type=jax.ShapeDtypeStruct((NSC, NPAD, DH), jnp.float32),
        mesh=mesh,
        scratch_types=[
            pltpu.VMEM((2, CE), jnp.int32),
            pltpu.VMEM((2, CE), jnp.int32),
            pltpu.VMEM((2, CE), jnp.int32),
            pltpu.VMEM((2, CE), jnp.float32),
            pltpu.VMEM((2, CE, DH), jnp.float32),
            pltpu.VMEM((2, CE, DH), jnp.float32),
            pltpu.VMEM_SHARED((NPAD, DH), jnp.float32),
            pltpu.SemaphoreType.DMA,
            pltpu.SemaphoreType.DMA,
        ],
    )
    return f(row, col, eid, hx3, ctab3, dis)


@jax.jit
def _deg_call(row):
    mesh = plsc.VectorSubcoreMesh(**_MESH)
    f = pl.kernel(
        _deg_body,
        out_type=jax.ShapeDtypeStruct((NSC, NPAD), jnp.float32),
        mesh=mesh,
        scratch_types=[
            pltpu.VMEM((EPT,), jnp.int32),
            pltpu.VMEM((EPT + 16,), jnp.float32),
            pltpu.VMEM((NPAD // NTILES,), jnp.float32),
            pltpu.VMEM_SHARED((NPAD,), jnp.float32),
        ],
    )
    return f(row)


def _padE(a, fill):
    a2 = a.reshape(NTILES, E // NTILES).astype(jnp.int32)
    return jnp.pad(a2, ((0, 0), (0, EPTP - E // NTILES)),
                   constant_values=fill).reshape(-1)


def _gcn_layer(h, row, col, eid, W, b, root, ctab, deg, dis):
    hx = h @ W + b
    hxp = jnp.zeros((NPAD, D), jnp.float32).at[:N].set(dis[:, None] * hx)
    hx3 = jnp.stack([hxp[:, :DH], hxp[:, DH:]])
    ctab3 = jnp.stack([ctab[:, :DH], ctab[:, DH:]])
    disp = jnp.zeros((NPAD,), jnp.float32).at[:N].set(dis)
    aggp = _edge_call(_padE(row, 0), _padE(col, N), _padE(eid, 0), hx3, ctab3, disp)
    agg = jnp.concatenate([aggp[0, :N], aggp[1, :N]], axis=1) * dis[:, None]
    return agg + jax.nn.relu(hx + root) / deg[:, None]


def kernel(x, edge_index, edge_attr, batch, atom_tab, bond_tab1, bond_tab2,
           W1, b1, root1, W2, b2, root2, Wg, bg):
    row = edge_index[0]
    col = edge_index[1]
    h = atom_tab[0][x[:, 0]]
    for i in range(1, 9):
        h = h + atom_tab[i][x[:, i]]
    degp = _deg_call(row)
    deg = degp[0, :N] + degp[1, :N] + 1.0
    dis = deg ** -0.5
    eid = (edge_attr[:, 0] + 8 * edge_attr[:, 1] + 64 * edge_attr[:, 2]).astype(jnp.int32)
    ids = jnp.arange(512, dtype=jnp.int32)
    def _ctab(bt):
        return bt[0][ids & 7] + bt[1][(ids >> 3) & 7] + bt[2][(ids >> 6) & 7]
    h = _gcn_layer(h, row, col, eid, W1, b1, root1, _ctab(bond_tab1), deg, dis)
    h = _gcn_layer(h, row, col, eid, W2, b2, root2, _ctab(bond_tab2), deg, dis)
    hg = jax.ops.segment_max(h, batch, num_segments=G)
    return hg @ Wg + bg


# speedup vs baseline: 4.6987x; 1.0439x over previous
import functools

import jax
import jax.numpy as jnp
from jax import lax
from jax.experimental import pallas as pl
from jax.experimental.pallas import tpu as pltpu
from jax.experimental.pallas import tpu_sc as plsc

N = 10000
E = 160000
D = 256
G = 64

NSC = 2
NTILES = 16
NW = NSC * NTILES
NPAD = 10240
EPT = E // NW

_MESH = dict(core_axis_name="c", subcore_axis_name="s")


def _fill(ref, start, count16, vec):
    def body(i, _):
        ref[pl.ds(start + i * 16, 16)] = vec
        return 0
    lax.fori_loop(0, count16, body, 0)


def _deg_body(row_hbm, out_hbm, idx_v, val_v, zb_v, deg_sh):
    c = lax.axis_index("c")
    s = lax.axis_index("s")
    w = c * NTILES + s
    zsl = NPAD // NTILES
    _fill(zb_v, 0, zsl // 16, jnp.zeros((16,), jnp.float32))
    pltpu.sync_copy(zb_v, deg_sh.at[pl.ds(s * zsl, zsl)])
    _fill(val_v, 0, EPT // 16 + 1, jnp.ones((16,), jnp.float32))
    pltpu.sync_copy(row_hbm.at[pl.ds(w * EPT, EPT)], idx_v)
    plsc.subcore_barrier()
    pltpu.sync_copy(val_v.at[pl.ds(0, EPT)], deg_sh.at[idx_v], add=True)
    plsc.subcore_barrier()
    pltpu.sync_copy(deg_sh.at[pl.ds(s * zsl, zsl)], out_hbm.at[c, pl.ds(s * zsl, zsl)])


CE = 80
EPTP = 10240
NCH = EPTP // CE
DH = D // NSC


NBUF = 4


def _edge_body(row_hbm, col_hbm, eid_hbm, hx_hbm, ctab_hbm, dis_hbm, out_hbm,
               rowb, colb, eidb, nrmb, db, agg_sh, sem0, sem1, sem2, sem3):
    c = lax.axis_index("c")
    s = lax.axis_index("s")
    sems = (sem0, sem1, sem2, sem3)
    zsl = NPAD // NTILES
    z16 = jnp.zeros((16,), jnp.float32)

    def zfill(t, _):
        for k in range(DH // 16):
            db[0, t, pl.ds(k * 16, 16)] = z16
        return 0
    lax.fori_loop(0, CE, zfill, 0)

    def zcp(t, _):
        pltpu.sync_copy(db.at[0], agg_sh.at[pl.ds(s * zsl + t * CE, CE)])
        return 0
    lax.fori_loop(0, zsl // CE, zcp, 0)
    plsc.subcore_barrier()

    def fire1(b, i):
        base = s * EPTP + i * CE
        pltpu.sync_copy(row_hbm.at[pl.ds(base, CE)], rowb.at[b])
        pltpu.sync_copy(col_hbm.at[pl.ds(base, CE)], colb.at[b])
        pltpu.sync_copy(eid_hbm.at[pl.ds(base, CE)], eidb.at[b])
        pltpu.async_copy(ctab_hbm.at[c].at[eidb.at[b]], db.at[b], sems[b])
        pltpu.async_copy(dis_hbm.at[rowb.at[b]], nrmb.at[b], sems[b])

    def wait1(b):
        pltpu.make_async_copy(ctab_hbm.at[c].at[eidb.at[b]], db.at[b], sems[b]).wait()
        pltpu.make_async_copy(dis_hbm.at[rowb.at[b]], nrmb.at[b], sems[b]).wait()

    def fire2(b):
        pltpu.async_copy(hx_hbm.at[c].at[rowb.at[b]], db.at[b], sems[b], add=True)

    def wait2(b):
        pltpu.make_async_copy(hx_hbm.at[c].at[rowb.at[b]], db.at[b], sems[b]).wait()

    def compute_scatter(b):
        def egrp(g, _):
            nv16 = nrmb[b, pl.ds(g * 16, 16)]
            for jj in range(16):
                j = g * 16 + jj
                nv = nv16[jj]
                for k in range(DH // 16):
                    sl = pl.ds(k * 16, 16)
                    db[b, j, sl] = nv * jnp.maximum(db[b, j, sl], 0.0)
            return 0
        lax.fori_loop(0, CE // 16, egrp, 0)
        pltpu.sync_copy(db.at[b], agg_sh.at[colb.at[b]], add=True)

    fire1(0, 0)
    wait1(0)
    fire2(0)
    fire1(1, 1)

    def group(g, _):
        i0 = NBUF * g
        for j in range(NBUF):
            i = i0 + j
            b = j
            b1 = (j + 1) % NBUF
            b2 = (j + 2) % NBUF
            wait2(b)

            @pl.when(i + 1 < NCH)
            def _():
                wait1(b1)
                fire2(b1)

            @pl.when(i + 2 < NCH)
            def _():
                fire1(b2, i + 2)
            compute_scatter(b)
        return 0

    lax.fori_loop(0, NCH // NBUF, group, 0)
    plsc.subcore_barrier()
    pltpu.sync_copy(agg_sh.at[pl.ds(s * zsl, zsl)],
                    out_hbm.at[c, pl.ds(s * zsl, zsl)])


@jax.jit
def _edge_call(row, col, eid, hx3, ctab3, dis):
    mesh = plsc.VectorSubcoreMesh(**_MESH)
    f = pl.kernel(
        _edge_body,
        out_type=jax.ShapeDtypeStruct((NSC, NPAD, DH), jnp.float32),
        mesh=mesh,
        scratch_types=[
            pltpu.VMEM((NBUF, CE), jnp.int32),
            pltpu.VMEM((NBUF, CE), jnp.int32),
            pltpu.VMEM((NBUF, CE), jnp.int32),
            pltpu.VMEM((NBUF, CE), jnp.float32),
            pltpu.VMEM((NBUF, CE, DH), jnp.float32),
            pltpu.VMEM_SHARED((NPAD, DH), jnp.float32),
            pltpu.SemaphoreType.DMA,
            pltpu.SemaphoreType.DMA,
            pltpu.SemaphoreType.DMA,
            pltpu.SemaphoreType.DMA,
        ],
    )
    return f(row, col, eid, hx3, ctab3, dis)


@jax.jit
def _deg_call(row):
    mesh = plsc.VectorSubcoreMesh(**_MESH)
    f = pl.kernel(
        _deg_body,
        out_type=jax.ShapeDtypeStruct((NSC, NPAD), jnp.float32),
        mesh=mesh,
        scratch_types=[
            pltpu.VMEM((EPT,), jnp.int32),
            pltpu.VMEM((EPT + 16,), jnp.float32),
            pltpu.VMEM((NPAD // NTILES,), jnp.float32),
            pltpu.VMEM_SHARED((NPAD,), jnp.float32),
        ],
    )
    return f(row)


def _padE(a, fill):
    a2 = a.reshape(NTILES, E // NTILES).astype(jnp.int32)
    return jnp.pad(a2, ((0, 0), (0, EPTP - E // NTILES)),
                   constant_values=fill).reshape(-1)


def _gcn_layer(h, row, col, eid, W, b, root, ctab, deg, dis):
    hx = h @ W + b
    hxp = jnp.zeros((NPAD, D), jnp.float32).at[:N].set(hx)
    hx3 = jnp.stack([hxp[:, :DH], hxp[:, DH:]])
    ctab3 = jnp.stack([ctab[:, :DH], ctab[:, DH:]])
    disp = jnp.zeros((NPAD,), jnp.float32).at[:N].set(dis)
    aggp = _edge_call(_padE(row, 0), _padE(col, N), _padE(eid, 0), hx3, ctab3, disp)
    agg = jnp.concatenate([aggp[0, :N], aggp[1, :N]], axis=1) * dis[:, None]
    return agg + jax.nn.relu(hx + root) / deg[:, None]


def kernel(x, edge_index, edge_attr, batch, atom_tab, bond_tab1, bond_tab2,
           W1, b1, root1, W2, b2, root2, Wg, bg):
    row = edge_index[0]
    col = edge_index[1]
    h = atom_tab[0][x[:, 0]]
    for i in range(1, 9):
        h = h + atom_tab[i][x[:, i]]
    degp = _deg_call(row)
    deg = degp[0, :N] + degp[1, :N] + 1.0
    dis = deg ** -0.5
    eid = (edge_attr[:, 0] + 8 * edge_attr[:, 1] + 64 * edge_attr[:, 2]).astype(jnp.int32)
    ids = jnp.arange(512, dtype=jnp.int32)
    def _ctab(bt):
        return bt[0][ids & 7] + bt[1][(ids >> 3) & 7] + bt[2][(ids >> 6) & 7]
    h = _gcn_layer(h, row, col, eid, W1, b1, root1, _ctab(bond_tab1), deg, dis)
    h = _gcn_layer(h, row, col, eid, W2, b2, root2, _ctab(bond_tab2), deg, dis)
    hg = jax.ops.segment_max(h, batch, num_segments=G)
    return hg @ Wg + bg
